# Initial kernel scaffold; baseline (speedup 1.0000x reference)
import jax
import jax.numpy as jnp
from jax.experimental import pallas as pl


def kernel(x, W1, g1, b1, W2, g2, b2, W3, g3, b3, W3t, g3t, b3t, W2t, g2t, b2t, Wout, km1_in, km1_out, km1_sp, km2_in, km2_out, km2_sp, km3_in, km3_out, km3_sp, km3t_in, km3t_out, km3t_sp, km2t_in, km2t_out, km2t_sp, n1, n2, n3):
    raise NotImplementedError("write your pallas kernel here")


# speedup vs baseline: 40.7365x; 40.7365x over previous
import jax
import jax.numpy as jnp
from jax import lax
from jax.experimental import pallas as pl
from jax.experimental.pallas import tpu as pltpu
from jax.experimental.pallas import tpu_sc as plsc

_B = 512
_K = 27
_R1 = 50176
_R2 = 27136
_R3 = 4608
_EPAD = 32768


def _cdiv(a, b):
    return (a + b - 1) // b


def _dense_first(x, w):
    np_, ci = x.shape
    o = w.shape[1]

    def body(x_ref, w_ref, y_ref):
        y_ref[...] = jnp.dot(x_ref[...], w_ref[...],
                             preferred_element_type=jnp.float32)

    return pl.pallas_call(
        body,
        grid=(np_ // _B,),
        in_specs=[
            pl.BlockSpec((_B, ci), lambda i: (i, 0)),
            pl.BlockSpec((ci, o), lambda i: (0, 0)),
        ],
        out_specs=pl.BlockSpec((_B, o), lambda i: (i, 0)),
        out_shape=jax.ShapeDtypeStruct((np_, o), jnp.float32),
    )(x, w)


def _bn_stats(a, gb, nn, c):
    _, r, _ = a.shape
    r8 = r // 8
    a8 = a.reshape(2, r8, 8 * c)
    fold = jnp.tile(jnp.eye(c, dtype=jnp.float32), (8, 1))

    def body(a_ref, gb_ref, n_ref, f_ref, ss_ref):
        m = a_ref[0, : r8 - 1, :] + a_ref[1, : r8 - 1, :]
        s = jnp.sum(m, axis=0, keepdims=True)
        q = jnp.sum(m * m, axis=0, keepdims=True)
        sq = jnp.dot(jnp.concatenate([s, q], axis=0), f_ref[...],
                     preferred_element_type=jnp.float32)
        n = n_ref[0, 0]
        mu = sq[0, :] / n
        var = sq[1, :] / n - mu * mu
        sc = gb_ref[0, :] * lax.rsqrt(var + 1e-5)
        ss_ref[0, :] = sc
        ss_ref[1, :] = gb_ref[1, :] - mu * sc

    return pl.pallas_call(
        body,
        grid=(1,),
        in_specs=[
            pl.BlockSpec((2, r8, 8 * c), lambda i: (0, 0, 0)),
            pl.BlockSpec((2, c), lambda i: (0, 0)),
            pl.BlockSpec((1, 1), lambda i: (0, 0)),
            pl.BlockSpec((8 * c, c), lambda i: (0, 0)),
        ],
        out_specs=pl.BlockSpec((2, c), lambda i: (0, 0)),
        out_shape=jax.ShapeDtypeStruct((2, c), jnp.float32),
    )(a8, gb, nn, fold)


def _dense_mid(a, ss, w):
    _, r, c = a.shape
    o = w.shape[1]

    def body(a_ref, ss_ref, w_ref, y_ref):
        xb = a_ref[0] + a_ref[1]
        xb = jnp.maximum(xb * ss_ref[0, :] + ss_ref[1, :], 0.0)
        y_ref[...] = jnp.dot(xb, w_ref[...],
                             preferred_element_type=jnp.float32)

    return pl.pallas_call(
        body,
        grid=(r // _B,),
        in_specs=[
            pl.BlockSpec((2, _B, c), lambda i: (0, i, 0)),
            pl.BlockSpec((2, c), lambda i: (0, 0)),
            pl.BlockSpec((c, o), lambda i: (0, 0)),
        ],
        out_specs=pl.BlockSpec((_B, o), lambda i: (i, 0)),
        out_shape=jax.ShapeDtypeStruct((r, o), jnp.float32),
    )(a, ss, w)


def _dense_two(aa, ssa, ab, ssb, wa, wb):
    _, r, ca = aa.shape
    cb = ab.shape[2]
    o = wa.shape[1]

    def body(aa_ref, sa_ref, ab_ref, sb_ref, wa_ref, wb_ref, y_ref):
        xa = aa_ref[0] + aa_ref[1]
        xa = jnp.maximum(xa * sa_ref[0, :] + sa_ref[1, :], 0.0)
        xb = ab_ref[0] + ab_ref[1]
        xb = xb * sb_ref[0, :] + sb_ref[1, :]
        y_ref[...] = (
            jnp.dot(xa, wa_ref[...], preferred_element_type=jnp.float32)
            + jnp.dot(xb, wb_ref[...], preferred_element_type=jnp.float32))

    return pl.pallas_call(
        body,
        grid=(r // _B,),
        in_specs=[
            pl.BlockSpec((2, _B, ca), lambda i: (0, i, 0)),
            pl.BlockSpec((2, ca), lambda i: (0, 0)),
            pl.BlockSpec((2, _B, cb), lambda i: (0, i, 0)),
            pl.BlockSpec((2, cb), lambda i: (0, 0)),
            pl.BlockSpec((ca, o), lambda i: (0, 0)),
            pl.BlockSpec((cb, o), lambda i: (0, 0)),
        ],
        out_specs=pl.BlockSpec((_B, o), lambda i: (i, 0)),
        out_shape=jax.ShapeDtypeStruct((r, o), jnp.float32),
    )(aa, ssa, ab, ssb, wa, wb)


def _sc_scatter(y2d, fidx2, mout2, zrow, r, c):
    ec = fidx2.shape[0]
    cpt = ec // 32
    ng = cpt // 8
    rps = r // 16

    mesh = plsc.VectorSubcoreMesh(core_axis_name="c", subcore_axis_name="s")

    def body(y_hbm, f_hbm, m_hbm, z_hbm, out_hbm, fbuf, mbuf, vals, acc, sem):
        cid = lax.axis_index("c")
        sid = lax.axis_index("s")
        tid = sid * 2 + cid
        pltpu.sync_copy(z_hbm, acc.at[pl.ds(sid * rps, rps)])
        plsc.subcore_barrier()

        base = tid * cpt

        def grp(g, carry):
            r0 = base + g * 8
            pltpu.sync_copy(f_hbm.at[pl.ds(r0, 8)], fbuf)
            pltpu.sync_copy(m_hbm.at[pl.ds(r0, 8)], mbuf)
            hs = [pltpu.async_copy(y_hbm.at[fbuf.at[j]], vals.at[j], sem)
                  for j in range(8)]
            for h in hs:
                h.wait()
            for j in range(8):
                pltpu.sync_copy(vals.at[j], acc.at[mbuf.at[j]], add=True)
            return carry

        lax.fori_loop(0, ng, grp, 0)
        plsc.subcore_barrier()
        pltpu.sync_copy(acc.at[pl.ds(sid * rps, rps)],
                        out_hbm.at[cid, pl.ds(sid * rps, rps)])

    f = pl.kernel(
        body,
        mesh=mesh,
        compiler_params=pltpu.CompilerParams(use_tc_tiling_on_sc=False),
        out_type=jax.ShapeDtypeStruct((2, r, c), jnp.float32),
        scratch_types=[
            pltpu.VMEM((8, 128), jnp.int32),
            pltpu.VMEM((8, 128), jnp.int32),
            pltpu.VMEM((8, 128, c), jnp.float32),
            pltpu.VMEM_SHARED((r, c), jnp.float32),
            pltpu.SemaphoreType.DMA,
        ])
    return f(y2d, fidx2, mout2, zrow)


def _edge_prep(m_in, m_out, sp, r):
    e = m_in.shape[0]
    epad = _cdiv(e, _EPAD) * _EPAD
    ke = (jnp.searchsorted(sp, jnp.arange(e, dtype=sp.dtype), side="right")
          - 1).astype(jnp.int32)
    fidx = m_in.astype(jnp.int32) * _K + ke
    mo = m_out.astype(jnp.int32)
    fidx = jnp.pad(fidx, (0, epad - e))
    mo = jnp.pad(mo, (0, epad - e), constant_values=r - 1)
    return fidx.reshape(epad // 128, 128), mo.reshape(epad // 128, 128)


def _wflat(w):
    return w.transpose(1, 0, 2).reshape(w.shape[1], _K * w.shape[2])


def kernel(x, W1, g1, b1, W2, g2, b2, W3, g3, b3, W3t, g3t, b3t,
           W2t, g2t, b2t, Wout, km1_in, km1_out, km1_sp, km2_in, km2_out,
           km2_sp, km3_in, km3_out, km3_sp, km3t_in, km3t_out, km3t_sp,
           km2t_in, km2t_out, km2t_sp, n1, n2, n3):
    f32 = jnp.float32
    nn1 = jnp.asarray(n1, f32).reshape(1, 1)
    nn2 = jnp.asarray(n2, f32).reshape(1, 1)
    nn3 = jnp.asarray(n3, f32).reshape(1, 1)
    gb1 = jnp.stack([g1, b1])
    gb2 = jnp.stack([g2, b2])
    gb3 = jnp.stack([g3, b3])
    gb3t = jnp.stack([g3t, b3t])
    gb2t = jnp.stack([g2t, b2t])

    xp = jnp.pad(x, ((0, _R1 - x.shape[0]), (0, 0)))

    y1 = _dense_first(xp, _wflat(W1))
    f1, m1 = _edge_prep(km1_in, km1_out, km1_sp, _R1)
    a1 = _sc_scatter(y1.reshape(_R1 * _K, 8), f1, m1,
                     jnp.zeros((_R1 // 16, 8), f32), _R1, 8)

    ss1 = _bn_stats(a1, gb1, nn1, 8)
    y2 = _dense_mid(a1, ss1, _wflat(W2))
    f2, m2 = _edge_prep(km2_in, km2_out, km2_sp, _R2)
    a2 = _sc_scatter(y2.reshape(_R1 * _K, 16), f2, m2,
                     jnp.zeros((_R2 // 16, 16), f32), _R2, 16)

    ss2 = _bn_stats(a2, gb2, nn2, 16)
    y3 = _dense_mid(a2, ss2, _wflat(W3))
    f3, m3 = _edge_prep(km3_in, km3_out, km3_sp, _R3)
    a3 = _sc_scatter(y3.reshape(_R2 * _K, 32), f3, m3,
                     jnp.zeros((_R3 // 16, 32), f32), _R3, 32)

    ss3 = _bn_stats(a3, gb3, nn3, 32)
    y3t = _dense_mid(a3, ss3, _wflat(W3t))
    f3t, m3t = _edge_prep(km3t_in, km3t_out, km3t_sp, _R2)
    a3t = _sc_scatter(y3t.reshape(_R3 * _K, 16), f3t, m3t,
                      jnp.zeros((_R2 // 16, 16), f32), _R2, 16)

    ss3t = _bn_stats(a3t, gb3t, nn2, 16)
    w2tf = _wflat(W2t)
    y2t = _dense_two(a3t, ss3t, a2, ss2, w2tf[:16], w2tf[16:])
    f2t, m2t = _edge_prep(km2t_in, km2t_out, km2t_sp, _R1)
    a2t = _sc_scatter(y2t.reshape(_R2 * _K, 16), f2t, m2t,
                      jnp.zeros((_R1 // 16, 16), f32), _R1, 16)

    ss2t = _bn_stats(a2t, gb2t, nn1, 16)
    out = _dense_two(a2t, ss2t, a1, ss1, Wout[:16], Wout[16:])
    return out[: x.shape[0]]


# speedup vs baseline: 68.3291x; 1.6773x over previous
import jax
import jax.numpy as jnp
from jax import lax
from jax.experimental import pallas as pl
from jax.experimental.pallas import tpu as pltpu
from jax.experimental.pallas import tpu_sc as plsc

_B = 1024
_K = 27
_R1 = 50176
_R2 = 27648
_R3M = 4096
_R3G = _R3M * _K
_EPAD = 32768


def _cdiv(a, b):
    return (a + b - 1) // b


def _dense_first(x, w, b=_B):
    np_, ci = x.shape
    o = w.shape[1]

    def body(x_ref, w_ref, y_ref):
        y_ref[...] = jnp.dot(x_ref[...], w_ref[...],
                             preferred_element_type=jnp.float32)

    return pl.pallas_call(
        body,
        grid=(np_ // b,),
        in_specs=[
            pl.BlockSpec((b, ci), lambda i: (i, 0)),
            pl.BlockSpec((ci, o), lambda i: (0, 0)),
        ],
        out_specs=pl.BlockSpec((b, o), lambda i: (i, 0)),
        out_shape=jax.ShapeDtypeStruct((np_, o), jnp.float32),
    )(x, w)


def _stats_core(m, gb_ref, n_ref, f_ref):
    s = jnp.sum(m, axis=0, keepdims=True)
    q = jnp.sum(m * m, axis=0, keepdims=True)
    sq = jnp.dot(jnp.concatenate([s, q], axis=0), f_ref[...],
                 preferred_element_type=jnp.float32)
    n = n_ref[0, 0]
    mu = sq[0, :] / n
    var = sq[1, :] / n - mu * mu
    sc = gb_ref[0, :] * lax.rsqrt(var + 1e-5)
    return sc, gb_ref[1, :] - mu * sc


def _stats_from_a8(a8_ref, gb_ref, n_ref, f_ref, r8):
    m = a8_ref[0, : r8 - 1, :] + a8_ref[1, : r8 - 1, :]
    return _stats_core(m, gb_ref, n_ref, f_ref)


def _dense_mid(a, gb, nn, w, b=_B, zero_last=False):
    _, r, c = a.shape
    o = w.shape[1]
    r8 = r // 8
    a8 = a.reshape(2, r8, 8 * c)
    fold = jnp.tile(jnp.eye(c, dtype=jnp.float32), (8, 1))

    def body(a_ref, a8_ref, gb_ref, n_ref, f_ref, w_ref, y_ref, ss_ref):
        i = pl.program_id(0)

        @pl.when(i == 0)
        def _():
            sc, sh = _stats_from_a8(a8_ref, gb_ref, n_ref, f_ref, r8)
            ss_ref[0, :] = sc
            ss_ref[1, :] = sh

        xb = a_ref[0] + a_ref[1]
        xb = jnp.maximum(xb * ss_ref[0, :] + ss_ref[1, :], 0.0)
        y_ref[...] = jnp.dot(xb, w_ref[...],
                             preferred_element_type=jnp.float32)
        if zero_last:
            @pl.when(i == r // b - 1)
            def _():
                y_ref[b - 8 :, :] = jnp.zeros((8, o), jnp.float32)

    y, ss = pl.pallas_call(
        body,
        grid=(r // b,),
        in_specs=[
            pl.BlockSpec((2, b, c), lambda i: (0, i, 0)),
            pl.BlockSpec((2, r8, 8 * c), lambda i: (0, 0, 0)),
            pl.BlockSpec((2, c), lambda i: (0, 0)),
            pl.BlockSpec((1, 1), lambda i: (0, 0)),
            pl.BlockSpec((8 * c, c), lambda i: (0, 0)),
            pl.BlockSpec((c, o), lambda i: (0, 0)),
        ],
        out_specs=[pl.BlockSpec((b, o), lambda i: (i, 0)),
                   pl.BlockSpec((2, c), lambda i: (0, 0))],
        out_shape=[jax.ShapeDtypeStruct((r, o), jnp.float32),
                   jax.ShapeDtypeStruct((2, c), jnp.float32)],
    )(a, a8, gb, nn, fold, w)
    return y, ss


def _dense_two(aa, gba, nna, ab, ssb, wa, wb, b=_B):
    _, r, ca = aa.shape
    cb = ab.shape[2]
    o = wa.shape[1]
    r8 = r // 8
    aa8 = aa.reshape(2, r8, 8 * ca)
    fold = jnp.tile(jnp.eye(ca, dtype=jnp.float32), (8, 1))

    def body(aa_ref, aa8_ref, gba_ref, na_ref, f_ref, ab_ref, sb_ref,
             wa_ref, wb_ref, y_ref, sa_ref):
        i = pl.program_id(0)

        @pl.when(i == 0)
        def _():
            sc, sh = _stats_from_a8(aa8_ref, gba_ref, na_ref, f_ref, r8)
            sa_ref[0, :] = sc
            sa_ref[1, :] = sh

        xa = aa_ref[0] + aa_ref[1]
        xa = jnp.maximum(xa * sa_ref[0, :] + sa_ref[1, :], 0.0)
        xb = ab_ref[0] + ab_ref[1]
        xb = xb * sb_ref[0, :] + sb_ref[1, :]
        y_ref[...] = (
            jnp.dot(xa, wa_ref[...], preferred_element_type=jnp.float32)
            + jnp.dot(xb, wb_ref[...], preferred_element_type=jnp.float32))

    return pl.pallas_call(
        body,
        grid=(r // b,),
        in_specs=[
            pl.BlockSpec((2, b, ca), lambda i: (0, i, 0)),
            pl.BlockSpec((2, r8, 8 * ca), lambda i: (0, 0, 0)),
            pl.BlockSpec((2, ca), lambda i: (0, 0)),
            pl.BlockSpec((1, 1), lambda i: (0, 0)),
            pl.BlockSpec((8 * ca, ca), lambda i: (0, 0)),
            pl.BlockSpec((2, b, cb), lambda i: (0, i, 0)),
            pl.BlockSpec((2, cb), lambda i: (0, 0)),
            pl.BlockSpec((ca, o), lambda i: (0, 0)),
            pl.BlockSpec((cb, o), lambda i: (0, 0)),
        ],
        out_specs=pl.BlockSpec((b, o), lambda i: (i, 0)),
        out_shape=jax.ShapeDtypeStruct((r, o), jnp.float32),
        scratch_shapes=[pltpu.VMEM((2, ca), jnp.float32)],
    )(aa, aa8, gba, nna, fold, ab, ssb, wa, wb)


def _mm_pair(a, w, b):
    _, r, ci = a.shape
    o = w.shape[1]

    def body(a_ref, w_ref, y_ref):
        y_ref[...] = jnp.dot(a_ref[0] + a_ref[1], w_ref[...],
                             preferred_element_type=jnp.float32)

    return pl.pallas_call(
        body,
        grid=(r // b,),
        in_specs=[
            pl.BlockSpec((2, b, ci), lambda i: (0, i, 0)),
            pl.BlockSpec((ci, o), lambda i: (0, 0)),
        ],
        out_specs=pl.BlockSpec((b, o), lambda i: (i, 0)),
        out_shape=jax.ShapeDtypeStruct((r, o), jnp.float32),
    )(a, w)


def _dense_mid1(a, gb, nn, w, b, drop8=True):
    r, c = a.shape
    o = w.shape[1]
    r8 = r // 8
    a8 = a.reshape(r8, 8 * c)
    fold = jnp.tile(jnp.eye(c, dtype=jnp.float32), (8, 1))

    def body(a_ref, a8_ref, gb_ref, n_ref, f_ref, w_ref, y_ref, ss_ref):
        i = pl.program_id(0)

        @pl.when(i == 0)
        def _():
            sc, sh = _stats_core(a8_ref[: r8 - 1 if drop8 else r8, :],
                                 gb_ref, n_ref, f_ref)
            ss_ref[0, :] = sc
            ss_ref[1, :] = sh

        xb = jnp.maximum(a_ref[...] * ss_ref[0, :] + ss_ref[1, :], 0.0)
        y_ref[...] = jnp.dot(xb, w_ref[...],
                             preferred_element_type=jnp.float32)

    y, _ = pl.pallas_call(
        body,
        grid=(r // b,),
        in_specs=[
            pl.BlockSpec((b, c), lambda i: (i, 0)),
            pl.BlockSpec((r8, 8 * c), lambda i: (0, 0)),
            pl.BlockSpec((2, c), lambda i: (0, 0)),
            pl.BlockSpec((1, 1), lambda i: (0, 0)),
            pl.BlockSpec((8 * c, c), lambda i: (0, 0)),
            pl.BlockSpec((c, o), lambda i: (0, 0)),
        ],
        out_specs=[pl.BlockSpec((b, o), lambda i: (i, 0)),
                   pl.BlockSpec((2, c), lambda i: (0, 0))],
        out_shape=[jax.ShapeDtypeStruct((r, o), jnp.float32),
                   jax.ShapeDtypeStruct((2, c), jnp.float32)],
    )(a, a8, gb, nn, fold, w)
    return y


def _sc_scatter(y2d, fidx2, mout2, zrow, r, c):
    ec = fidx2.shape[0]
    cpt = ec // 32
    ng = cpt // 8
    rps = r // 16

    mesh = plsc.VectorSubcoreMesh(core_axis_name="c", subcore_axis_name="s")

    def body(y_hbm, f_hbm, m_hbm, z_hbm, out_hbm, fbuf, mbuf, vals, acc, sem):
        cid = lax.axis_index("c")
        sid = lax.axis_index("s")
        tid = sid * 2 + cid
        pltpu.sync_copy(z_hbm, acc.at[pl.ds(sid * rps, rps)])
        plsc.subcore_barrier()

        base = tid * cpt

        def grp(g, carry):
            r0 = (base + g * 8) * 128
            pltpu.sync_copy(f_hbm.at[pl.ds(r0, 1024)], fbuf)
            pltpu.sync_copy(m_hbm.at[pl.ds(r0, 1024)], mbuf)
            pltpu.async_copy(y_hbm.at[fbuf], vals, sem).wait()
            pltpu.sync_copy(vals, acc.at[mbuf], add=True)
            return carry

        lax.fori_loop(0, ng, grp, 0)
        plsc.subcore_barrier()
        pltpu.sync_copy(acc.at[pl.ds(sid * rps, rps)],
                        out_hbm.at[cid, pl.ds(sid * rps, rps)])

    f = pl.kernel(
        body,
        mesh=mesh,
        compiler_params=pltpu.CompilerParams(use_tc_tiling_on_sc=False),
        out_type=jax.ShapeDtypeStruct((2, r, c), jnp.float32),
        scratch_types=[
            pltpu.VMEM((1024,), jnp.int32),
            pltpu.VMEM((1024,), jnp.int32),
            pltpu.VMEM((1024, c), jnp.float32),
            pltpu.VMEM_SHARED((r, c), jnp.float32),
            pltpu.SemaphoreType.DMA,
        ])
    return f(y2d, fidx2.reshape(-1), mout2.reshape(-1), zrow)


def _edge_prep(m_in, m_out, sp, r, gather_first=False,
               pad_fidx=0, pad_mo=None):
    e = m_in.shape[0]
    epad = _cdiv(e, _EPAD) * _EPAD
    if pad_mo is None:
        pad_mo = r - 1
    eidx = jnp.arange(e, dtype=jnp.int32)
    ke = jnp.sum(eidx[:, None] >= sp[None, 1:_K].astype(jnp.int32),
                 axis=1).astype(jnp.int32)
    if gather_first:
        fidx = m_in.astype(jnp.int32)
        mo = m_out.astype(jnp.int32) * _K + ke
    else:
        fidx = m_in.astype(jnp.int32) * _K + ke
        mo = m_out.astype(jnp.int32)
    fidx = jnp.pad(fidx, (0, epad - e), constant_values=pad_fidx)
    mo = jnp.pad(mo, (0, epad - e), constant_values=pad_mo)
    return fidx.reshape(epad // 128, 128), mo.reshape(epad // 128, 128)


def _wflat(w):
    return w.transpose(1, 0, 2).reshape(w.shape[1], _K * w.shape[2])


def kernel(x, W1, g1, b1, W2, g2, b2, W3, g3, b3, W3t, g3t, b3t,
           W2t, g2t, b2t, Wout, km1_in, km1_out, km1_sp, km2_in, km2_out,
           km2_sp, km3_in, km3_out, km3_sp, km3t_in, km3t_out, km3t_sp,
           km2t_in, km2t_out, km2t_sp, n1, n2, n3):
    f32 = jnp.float32
    nn1 = jnp.asarray(n1, f32).reshape(1, 1)
    nn2 = jnp.asarray(n2, f32).reshape(1, 1)
    nn3 = jnp.asarray(n3, f32).reshape(1, 1)
    gb1 = jnp.stack([g1, b1])
    gb2 = jnp.stack([g2, b2])
    gb3 = jnp.stack([g3, b3])
    gb3t = jnp.stack([g3t, b3t])
    gb2t = jnp.stack([g2t, b2t])

    xp = jnp.pad(x, ((0, _R1 - x.shape[0]), (0, 0)))

    y1 = _dense_first(xp, _wflat(W1))
    f1, m1 = _edge_prep(km1_in, km1_out, km1_sp, _R1)
    a1 = _sc_scatter(y1.reshape(_R1 * _K, 8), f1, m1,
                     jnp.zeros((_R1 // 16, 8), f32), _R1, 8)

    y2, ss1 = _dense_mid(a1, gb1, nn1, _wflat(W2))
    f2, m2 = _edge_prep(km2_in, km2_out, km2_sp, _R2)
    a2 = _sc_scatter(y2.reshape(_R1 * _K, 16), f2, m2,
                     jnp.zeros((_R2 // 16, 16), f32), _R2, 16)

    x2, ss2 = _dense_mid(a2, gb2, nn2, jnp.eye(16, dtype=f32),
                         zero_last=True)
    f3, m3 = _edge_prep(km3_in, km3_out, km3_sp, _R3G, gather_first=True,
                        pad_fidx=_R2 - 1, pad_mo=0)
    s3 = _sc_scatter(x2, f3, m3, jnp.zeros((_R3G // 16, 16), f32), _R3G, 16)
    a3 = _mm_pair(s3.reshape(2, _R3M, _K * 16), W3.reshape(_K * 16, 32), 1024)

    y3t = _dense_mid1(a3, gb3, nn3, _wflat(W3t), b=1024, drop8=False)
    f3t, m3t = _edge_prep(km3t_in, km3t_out, km3t_sp, _R2)
    a3t = _sc_scatter(y3t.reshape(_R3M * _K, 16), f3t, m3t,
                      jnp.zeros((_R2 // 16, 16), f32), _R2, 16)

    w2tf = _wflat(W2t)
    y2t = _dense_two(a3t, gb3t, nn2, a2, ss2, w2tf[:16], w2tf[16:])
    f2t, m2t = _edge_prep(km2t_in, km2t_out, km2t_sp, _R1)
    a2t = _sc_scatter(y2t.reshape(_R2 * _K, 16), f2t, m2t,
                      jnp.zeros((_R1 // 16, 16), f32), _R1, 16)

    out = _dense_two(a2t, gb2t, nn1, a1, ss1, Wout[:16], Wout[16:])
    return out[: x.shape[0]]
